# baseline (device time: 892659 ns/iter reference)
import jax
import jax.numpy as jnp
from jax import lax
from jax.experimental import pallas as pl
from jax.experimental.pallas import tpu as pltpu

N_DEV = 32
M = 4096
N = 2048
CHUNK = M // N_DEV
HALF = N // 2
N_HOPS = 2 * (N_DEV - 1)


def kernel(x, w_mat, scale_x, scale_w):
    m, k_per = x.shape
    k_per2, n = w_mat.shape
    assert m == M and n == N and k_per == k_per2

    s = (scale_x * scale_w).reshape(1, 1)

    def body(x_ref, w_ref, s_ref, out_ref,
             comm_r, comm_l, ss_r, rs_r, ss_l, rs_l, credit_r, credit_l):
        my = lax.axis_index("i")
        left = jnp.remainder(my - 1, N_DEV)
        right = jnp.remainder(my + 1, N_DEV)

        out_ref[...] = jnp.dot(
            x_ref[...].astype(jnp.bfloat16),
            w_ref[...].astype(jnp.bfloat16),
            preferred_element_type=jnp.float32,
        )

        barrier_sem = pltpu.get_barrier_semaphore()
        for nbr in (left, right):
            pl.semaphore_signal(
                barrier_sem, inc=1,
                device_id=(nbr,), device_id_type=pl.DeviceIdType.MESH,
            )
        pl.semaphore_wait(barrier_sem, 2)

        rdmas_r = []
        rdmas_l = []
        for t in range(N_HOPS):
            slot = t % 2
            if t >= 2:
                pl.semaphore_wait(credit_r, 1)
                pl.semaphore_wait(credit_l, 1)
                rdmas_r[t - 2].wait_send()
                rdmas_l[t - 2].wait_send()

            cs_r = jnp.remainder(my - t, N_DEV)
            cr_r = jnp.remainder(my - t - 1, N_DEV)
            cs_l = jnp.remainder(my + t, N_DEV)
            cr_l = jnp.remainder(my + t + 1, N_DEV)

            rr = pltpu.make_async_remote_copy(
                src_ref=out_ref.at[pl.ds(cs_r * CHUNK, CHUNK), pl.ds(0, HALF)],
                dst_ref=comm_r.at[slot],
                send_sem=ss_r.at[slot],
                recv_sem=rs_r.at[slot],
                device_id=(right,),
                device_id_type=pl.DeviceIdType.MESH,
            )
            rl = pltpu.make_async_remote_copy(
                src_ref=out_ref.at[pl.ds(cs_l * CHUNK, CHUNK), pl.ds(HALF, HALF)],
                dst_ref=comm_l.at[slot],
                send_sem=ss_l.at[slot],
                recv_sem=rs_l.at[slot],
                device_id=(left,),
                device_id_type=pl.DeviceIdType.MESH,
            )
            rr.start()
            rl.start()
            rdmas_r.append(rr)
            rdmas_l.append(rl)

            rr.wait_recv()
            rl.wait_recv()
            if t < N_DEV - 1:
                out_ref[pl.ds(cr_r * CHUNK, CHUNK), pl.ds(0, HALF)] = (
                    out_ref[pl.ds(cr_r * CHUNK, CHUNK), pl.ds(0, HALF)]
                    + comm_r[slot]
                )
                out_ref[pl.ds(cr_l * CHUNK, CHUNK), pl.ds(HALF, HALF)] = (
                    out_ref[pl.ds(cr_l * CHUNK, CHUNK), pl.ds(HALF, HALF)]
                    + comm_l[slot]
                )
            else:
                out_ref[pl.ds(cr_r * CHUNK, CHUNK), pl.ds(0, HALF)] = comm_r[slot]
                out_ref[pl.ds(cr_l * CHUNK, CHUNK), pl.ds(HALF, HALF)] = comm_l[slot]

            if t < N_HOPS - 2:
                pl.semaphore_signal(
                    credit_r, inc=1,
                    device_id=(left,), device_id_type=pl.DeviceIdType.MESH,
                )
                pl.semaphore_signal(
                    credit_l, inc=1,
                    device_id=(right,), device_id_type=pl.DeviceIdType.MESH,
                )

            if t == N_DEV - 2:
                sc = s_ref[0, 0]
                g_r = jnp.remainder(my + 1, N_DEV)
                y = out_ref[pl.ds(g_r * CHUNK, CHUNK), pl.ds(0, HALF)] * sc
                out_ref[pl.ds(g_r * CHUNK, CHUNK), pl.ds(0, HALF)] = (
                    y / (1.0 + jnp.exp(-jnp.clip(y, -60.0, 60.0)))
                )
                g_l = jnp.remainder(my - 1, N_DEV)
                y = out_ref[pl.ds(g_l * CHUNK, CHUNK), pl.ds(HALF, HALF)] * sc
                out_ref[pl.ds(g_l * CHUNK, CHUNK), pl.ds(HALF, HALF)] = (
                    y / (1.0 + jnp.exp(-jnp.clip(y, -60.0, 60.0)))
                )

        for t in (N_HOPS - 2, N_HOPS - 1):
            rdmas_r[t].wait_send()
            rdmas_l[t].wait_send()

    return pl.pallas_call(
        body,
        out_shape=jax.ShapeDtypeStruct((M, N), jnp.float32),
        in_specs=[
            pl.BlockSpec(memory_space=pltpu.VMEM),
            pl.BlockSpec(memory_space=pltpu.VMEM),
            pl.BlockSpec(memory_space=pltpu.SMEM),
        ],
        out_specs=pl.BlockSpec(memory_space=pltpu.VMEM),
        scratch_shapes=[
            pltpu.VMEM((2, CHUNK, HALF), jnp.float32),
            pltpu.VMEM((2, CHUNK, HALF), jnp.float32),
            pltpu.SemaphoreType.DMA((2,)),
            pltpu.SemaphoreType.DMA((2,)),
            pltpu.SemaphoreType.DMA((2,)),
            pltpu.SemaphoreType.DMA((2,)),
            pltpu.SemaphoreType.REGULAR,
            pltpu.SemaphoreType.REGULAR,
        ],
        compiler_params=pltpu.CompilerParams(
            collective_id=0, vmem_limit_bytes=100 * 1024 * 1024
        ),
    )(x, w_mat, s)


# device time: 512717 ns/iter; 1.7410x vs baseline; 1.7410x over previous
import jax
import jax.numpy as jnp
from jax import lax
from jax.experimental import pallas as pl
from jax.experimental.pallas import tpu as pltpu

N_DEV = 32
M = 4096
N = 2048
HALF_M = M // 2
CHUNK = HALF_M // N_DEV
N_HOPS = 2 * (N_DEV - 1)

_PLANE_IDX = {(0, 0): 0, (1, 0): 1, (1, 1): 2, (0, 1): 3,
              (0, 2): 4, (1, 2): 5, (1, 3): 6, (0, 3): 7}


def _hamiltonian_perm():
    cycle = []
    for y in range(4):
        zs = range(4) if y % 2 == 0 else range(3, -1, -1)
        cycle.extend((0, y, z) for z in zs)
    for y in range(3, -1, -1):
        zs = range(4) if y % 2 == 1 else range(3, -1, -1)
        cycle.extend((1, y, z) for z in zs)
    assert len(set(cycle)) == N_DEV
    return [8 * z + _PLANE_IDX[(x, y)] for (x, y, z) in cycle]


_PERM = _hamiltonian_perm()
_POS = [0] * N_DEV
_NEXT = [0] * N_DEV
_PREV = [0] * N_DEV
for _p, _l in enumerate(_PERM):
    _POS[_l] = _p
    _NEXT[_l] = _PERM[(_p + 1) % N_DEV]
    _PREV[_l] = _PERM[(_p - 1) % N_DEV]


def kernel(x, w_mat, scale_x, scale_w):
    m, k_per = x.shape
    k_per2, n = w_mat.shape
    assert m == M and n == N and k_per == k_per2

    s = (scale_x * scale_w).reshape(1, 1)

    my = lax.axis_index("i")
    ring = jnp.stack([
        jnp.array(_POS, jnp.int32)[my],
        jnp.array(_NEXT, jnp.int32)[my],
        jnp.array(_PREV, jnp.int32)[my],
    ]).reshape(1, 3)

    def body(x_ref, w_ref, s_ref, ring_ref, out_ref,
             comm_r, comm_l, ss_r, rs_r, ss_l, rs_l, credit_r, credit_l):
        pos = ring_ref[0, 0]
        nxt = ring_ref[0, 1]
        prv = ring_ref[0, 2]

        out_ref[...] = jnp.dot(
            x_ref[...].astype(jnp.bfloat16),
            w_ref[...].astype(jnp.bfloat16),
            preferred_element_type=jnp.float32,
        )

        barrier_sem = pltpu.get_barrier_semaphore()
        for nbr in (prv, nxt):
            pl.semaphore_signal(
                barrier_sem, inc=1,
                device_id=(nbr,), device_id_type=pl.DeviceIdType.MESH,
            )
        pl.semaphore_wait(barrier_sem, 2)

        rdmas_r = []
        rdmas_l = []
        for t in range(N_HOPS):
            slot = t % 2
            if t >= 2:
                pl.semaphore_wait(credit_r, 1)
                pl.semaphore_wait(credit_l, 1)
                rdmas_r[t - 2].wait_send()
                rdmas_l[t - 2].wait_send()

            cs_r = jnp.remainder(pos - t, N_DEV)
            cr_r = jnp.remainder(pos - t - 1, N_DEV)
            cs_l = jnp.remainder(pos + t, N_DEV)
            cr_l = jnp.remainder(pos + t + 1, N_DEV)

            rr = pltpu.make_async_remote_copy(
                src_ref=out_ref.at[pl.ds(cs_r * CHUNK, CHUNK), :],
                dst_ref=comm_r.at[slot],
                send_sem=ss_r.at[slot],
                recv_sem=rs_r.at[slot],
                device_id=(nxt,),
                device_id_type=pl.DeviceIdType.MESH,
            )
            rl = pltpu.make_async_remote_copy(
                src_ref=out_ref.at[pl.ds(HALF_M + cs_l * CHUNK, CHUNK), :],
                dst_ref=comm_l.at[slot],
                send_sem=ss_l.at[slot],
                recv_sem=rs_l.at[slot],
                device_id=(prv,),
                device_id_type=pl.DeviceIdType.MESH,
            )
            rr.start()
            rl.start()
            rdmas_r.append(rr)
            rdmas_l.append(rl)

            rr.wait_recv()
            rl.wait_recv()
            if t < N_DEV - 1:
                out_ref[pl.ds(cr_r * CHUNK, CHUNK), :] = (
                    out_ref[pl.ds(cr_r * CHUNK, CHUNK), :] + comm_r[slot]
                )
                out_ref[pl.ds(HALF_M + cr_l * CHUNK, CHUNK), :] = (
                    out_ref[pl.ds(HALF_M + cr_l * CHUNK, CHUNK), :] + comm_l[slot]
                )
            else:
                out_ref[pl.ds(cr_r * CHUNK, CHUNK), :] = comm_r[slot]
                out_ref[pl.ds(HALF_M + cr_l * CHUNK, CHUNK), :] = comm_l[slot]

            if t < N_HOPS - 2:
                pl.semaphore_signal(
                    credit_r, inc=1,
                    device_id=(prv,), device_id_type=pl.DeviceIdType.MESH,
                )
                pl.semaphore_signal(
                    credit_l, inc=1,
                    device_id=(nxt,), device_id_type=pl.DeviceIdType.MESH,
                )

            if t == N_DEV - 2:
                sc = s_ref[0, 0]
                g_r = jnp.remainder(pos + 1, N_DEV)
                y = out_ref[pl.ds(g_r * CHUNK, CHUNK), :] * sc
                out_ref[pl.ds(g_r * CHUNK, CHUNK), :] = (
                    y / (1.0 + jnp.exp(-jnp.clip(y, -60.0, 60.0)))
                )
                g_l = jnp.remainder(pos - 1, N_DEV)
                y = out_ref[pl.ds(HALF_M + g_l * CHUNK, CHUNK), :] * sc
                out_ref[pl.ds(HALF_M + g_l * CHUNK, CHUNK), :] = (
                    y / (1.0 + jnp.exp(-jnp.clip(y, -60.0, 60.0)))
                )

        for t in (N_HOPS - 2, N_HOPS - 1):
            rdmas_r[t].wait_send()
            rdmas_l[t].wait_send()

    return pl.pallas_call(
        body,
        out_shape=jax.ShapeDtypeStruct((M, N), jnp.float32),
        in_specs=[
            pl.BlockSpec(memory_space=pltpu.VMEM),
            pl.BlockSpec(memory_space=pltpu.VMEM),
            pl.BlockSpec(memory_space=pltpu.SMEM),
            pl.BlockSpec(memory_space=pltpu.SMEM),
        ],
        out_specs=pl.BlockSpec(memory_space=pltpu.VMEM),
        scratch_shapes=[
            pltpu.VMEM((2, CHUNK, N), jnp.float32),
            pltpu.VMEM((2, CHUNK, N), jnp.float32),
            pltpu.SemaphoreType.DMA((2,)),
            pltpu.SemaphoreType.DMA((2,)),
            pltpu.SemaphoreType.DMA((2,)),
            pltpu.SemaphoreType.DMA((2,)),
            pltpu.SemaphoreType.REGULAR,
            pltpu.SemaphoreType.REGULAR,
        ],
        compiler_params=pltpu.CompilerParams(
            collective_id=0, vmem_limit_bytes=100 * 1024 * 1024
        ),
    )(x, w_mat, s, ring)
